# Initial kernel scaffold; baseline (speedup 1.0000x reference)
import functools

import jax
import jax.numpy as jnp
from jax import lax
from jax.experimental import pallas as pl
from jax.experimental.pallas import tpu as pltpu
from jax.experimental.pallas import tpu_sc as plsc

N = 10000
D = 128
E = 320000
NC = 2
NS = 16
NW = NC * NS
C = 128
CH = 79
EPAD = NW * CH * C
NPAD = 10240
RPT = NPAD // NS
DEGW = 16

_mesh = plsc.VectorSubcoreMesh(core_axis_name="c", subcore_axis_name="s")


def _sc_body(with_deg, *refs):
    if with_deg:
        (x_hbm, src_hbm, dst_hbm, agg_out, deg_out,
         src_v, dst_v, rows_v, ones_v, zrow_v, agg_s, deg_s) = refs
    else:
        (x_hbm, src_hbm, dst_hbm, agg_out,
         src_v, dst_v, rows_v, zrow_v, agg_s) = refs
        ones_v = deg_out = deg_s = None

    cid = lax.axis_index("c")
    sid = lax.axis_index("s")
    wid = sid * NC + cid

    zf = jnp.zeros((16,), jnp.float32)
    for r in range(16):
        for g in range(D // 16):
            zrow_v[r, pl.ds(g * 16, 16)] = zf
    if with_deg:
        of = jnp.ones((16,), jnp.float32)
        for r in range(C):
            ones_v[r, :] = of

    def zero_body(k, _):
        base = sid * RPT + k * 16
        pltpu.sync_copy(zrow_v, agg_s.at[pl.ds(base, 16)])
        if with_deg:
            pltpu.sync_copy(zrow_v.at[:, pl.ds(0, DEGW)],
                            deg_s.at[pl.ds(base, 16)])
        return 0

    lax.fori_loop(0, RPT // 16, zero_body, 0)
    plsc.subcore_barrier()

    pltpu.sync_copy(src_hbm.at[wid], src_v)
    pltpu.sync_copy(dst_hbm.at[wid], dst_v)

    def edge_body(j, _):
        pltpu.sync_copy(x_hbm.at[src_v.at[j]], rows_v)
        pltpu.sync_copy(rows_v, agg_s.at[dst_v.at[j]], add=True)
        if with_deg:
            pltpu.sync_copy(ones_v, deg_s.at[dst_v.at[j]], add=True)
        return 0

    lax.fori_loop(0, CH, edge_body, 0)
    plsc.subcore_barrier()

    base = sid * RPT
    pltpu.sync_copy(agg_s.at[pl.ds(base, RPT)],
                    agg_out.at[cid, pl.ds(base, RPT)])
    if with_deg:
        pltpu.sync_copy(deg_s.at[pl.ds(base, RPT)],
                        deg_out.at[cid, pl.ds(base, RPT)])


def _make_sc_agg(with_deg):
    if with_deg:
        out_type = (jax.ShapeDtypeStruct((NC, NPAD, D), jnp.float32),
                    jax.ShapeDtypeStruct((NC, NPAD, DEGW), jnp.float32))
    else:
        out_type = jax.ShapeDtypeStruct((NC, NPAD, D), jnp.float32)
    scratch = [
        pltpu.VMEM((CH, C), jnp.int32),
        pltpu.VMEM((CH, C), jnp.int32),
        pltpu.VMEM((C, D), jnp.float32),
    ]
    if with_deg:
        scratch.append(pltpu.VMEM((C, DEGW), jnp.float32))
    scratch.append(pltpu.VMEM((16, D), jnp.float32))
    scratch.append(pltpu.VMEM_SHARED((NPAD, D), jnp.float32))
    if with_deg:
        scratch.append(pltpu.VMEM_SHARED((NPAD, DEGW), jnp.float32))
    return pl.kernel(
        functools.partial(_sc_body, with_deg),
        out_type=out_type,
        mesh=_mesh,
        scratch_types=scratch,
    )


_sc_agg_deg = _make_sc_agg(True)
_sc_agg = _make_sc_agg(False)

_BR = 1024


def _tc_body(relu, x_ref, p_ref, d_ref, ws_ref, wn_ref, b_ref, o_ref):
    agg = p_ref[0] + p_ref[1]
    deg = d_ref[0, :, 0:1] + d_ref[1, :, 0:1]
    hn = agg / jnp.maximum(deg, 1.0)
    acc = jnp.dot(x_ref[...], ws_ref[...], preferred_element_type=jnp.float32)
    acc = acc + jnp.dot(hn, wn_ref[...], preferred_element_type=jnp.float32)
    acc = acc + b_ref[...]
    if relu:
        acc = jnp.maximum(acc, 0.0)
    o_ref[...] = acc


def _tc_layer(x, aggp, degp, w_self, w_neigh, b, relu):
    fn = pl.pallas_call(
        functools.partial(_tc_body, relu),
        grid=(NPAD // _BR,),
        in_specs=[
            pl.BlockSpec((_BR, D), lambda i: (i, 0)),
            pl.BlockSpec((NC, _BR, D), lambda i: (0, i, 0)),
            pl.BlockSpec((NC, _BR, DEGW), lambda i: (0, i, 0)),
            pl.BlockSpec((D, D), lambda i: (0, 0)),
            pl.BlockSpec((D, D), lambda i: (0, 0)),
            pl.BlockSpec((1, D), lambda i: (0, 0)),
        ],
        out_specs=pl.BlockSpec((_BR, D), lambda i: (i, 0)),
        out_shape=jax.ShapeDtypeStruct((NPAD, D), jnp.float32),
    )
    return fn(x, aggp, degp, w_self, w_neigh, b.reshape(1, D))


def kernel(in_feat, W_self1, W_neigh1, b1, W_self2, W_neigh2, b2, edge_index):
    src = edge_index[0].astype(jnp.int32)
    dst = edge_index[1].astype(jnp.int32)
    pad = EPAD - E
    src_r = jnp.concatenate([src, jnp.zeros((pad,), jnp.int32)]).reshape(NW, CH, C)
    dst_r = jnp.concatenate([dst, jnp.full((pad,), N, jnp.int32)]).reshape(NW, CH, C)
    x_pad = jnp.pad(in_feat, ((0, NPAD - N), (0, 0)))

    aggp1, degp1 = _sc_agg_deg(x_pad, src_r, dst_r)
    h1 = _tc_layer(x_pad, aggp1, degp1, W_self1, W_neigh1, b1, relu=True)
    aggp2 = _sc_agg(h1, src_r, dst_r)
    out = _tc_layer(h1, aggp2, degp1, W_self2, W_neigh2, b2, relu=False)
    return out[:N]


# speedup vs baseline: 4.8524x; 4.8524x over previous
import functools

import jax
import jax.numpy as jnp
from jax import lax
from jax.experimental import pallas as pl
from jax.experimental.pallas import tpu as pltpu
from jax.experimental.pallas import tpu_sc as plsc

N = 10000
D = 128
E = 320000
NC = 2
NS = 16
NW = NC * NS
C = 128
CH = 79
EPAD = NW * CH * C
NPAD = 10240
RPT = NPAD // NS
DEGW = 16

_mesh = plsc.VectorSubcoreMesh(core_axis_name="c", subcore_axis_name="s")


def _sc_agg_body(x_hbm, src_hbm, dst_hbm, agg_out,
                 src_v, dst_v, rows_v, zrow_v, agg_s):
    cid = lax.axis_index("c")
    sid = lax.axis_index("s")
    wid = sid * NC + cid

    zf = jnp.zeros((16,), jnp.float32)
    for r in range(16):
        for g in range(D // 16):
            zrow_v[r, pl.ds(g * 16, 16)] = zf

    def zero_body(k, _):
        pltpu.sync_copy(zrow_v, agg_s.at[pl.ds(sid * RPT + k * 16, 16)])
        return 0

    lax.fori_loop(0, RPT // 16, zero_body, 0)
    plsc.subcore_barrier()

    pltpu.sync_copy(src_hbm.at[wid], src_v)
    pltpu.sync_copy(dst_hbm.at[wid], dst_v)

    def edge_body(j, _):
        pltpu.sync_copy(x_hbm.at[src_v.at[j]], rows_v)
        pltpu.sync_copy(rows_v, agg_s.at[dst_v.at[j]], add=True)
        return 0

    lax.fori_loop(0, CH, edge_body, 0)
    plsc.subcore_barrier()

    base = sid * RPT
    pltpu.sync_copy(agg_s.at[pl.ds(base, RPT)],
                    agg_out.at[cid, pl.ds(base, RPT)])


_sc_agg = pl.kernel(
    _sc_agg_body,
    out_type=jax.ShapeDtypeStruct((NC, NPAD, D), jnp.float32),
    mesh=_mesh,
    scratch_types=[
        pltpu.VMEM((CH, C), jnp.int32),
        pltpu.VMEM((CH, C), jnp.int32),
        pltpu.VMEM((C, D), jnp.float32),
        pltpu.VMEM((16, D), jnp.float32),
        pltpu.VMEM_SHARED((NPAD, D), jnp.float32),
    ],
)


def _sc_deg_body(dst_hbm, deg_out, dst_v, ones_v, deg_s):
    cid = lax.axis_index("c")
    sid = lax.axis_index("s")
    wid = sid * NC + cid

    zf = jnp.zeros((16,), jnp.float32)
    of = jnp.ones((16,), jnp.float32)
    for r in range(16):
        for g in range(D // 16):
            ones_v[r, pl.ds(g * 16, 16)] = zf
    for r in range(16, 16 + C):
        for g in range(D // 16):
            ones_v[r, pl.ds(g * 16, 16)] = of

    def zero_body(k, _):
        pltpu.sync_copy(ones_v.at[pl.ds(0, 16)],
                        deg_s.at[pl.ds(sid * RPT + k * 16, 16)])
        return 0

    lax.fori_loop(0, RPT // 16, zero_body, 0)
    plsc.subcore_barrier()

    pltpu.sync_copy(dst_hbm.at[wid], dst_v)

    def edge_body(j, _):
        pltpu.sync_copy(ones_v.at[pl.ds(16, C)], deg_s.at[dst_v.at[j]],
                        add=True)
        return 0

    lax.fori_loop(0, CH, edge_body, 0)
    plsc.subcore_barrier()

    base = sid * RPT
    pltpu.sync_copy(deg_s.at[pl.ds(base, RPT)],
                    deg_out.at[cid, pl.ds(base, RPT)])


_sc_deg = pl.kernel(
    _sc_deg_body,
    out_type=jax.ShapeDtypeStruct((NC, NPAD, D), jnp.float32),
    mesh=_mesh,
    scratch_types=[
        pltpu.VMEM((CH, C), jnp.int32),
        pltpu.VMEM((16 + C, D), jnp.float32),
        pltpu.VMEM_SHARED((NPAD, D), jnp.float32),
    ],
)

_BR = 1024


def _tc_body(relu, x_ref, p_ref, d_ref, ws_ref, wn_ref, b_ref, o_ref):
    agg = p_ref[0] + p_ref[1]
    deg = d_ref[0, :, 0:1] + d_ref[1, :, 0:1]
    hn = agg / jnp.maximum(deg, 1.0)
    acc = jnp.dot(x_ref[...], ws_ref[...], preferred_element_type=jnp.float32)
    acc = acc + jnp.dot(hn, wn_ref[...], preferred_element_type=jnp.float32)
    acc = acc + b_ref[...]
    if relu:
        acc = jnp.maximum(acc, 0.0)
    o_ref[...] = acc


def _tc_layer(x, aggp, degp, w_self, w_neigh, b, relu):
    fn = pl.pallas_call(
        functools.partial(_tc_body, relu),
        grid=(NPAD // _BR,),
        in_specs=[
            pl.BlockSpec((_BR, D), lambda i: (i, 0)),
            pl.BlockSpec((NC, _BR, D), lambda i: (0, i, 0)),
            pl.BlockSpec((NC, _BR, D), lambda i: (0, i, 0)),
            pl.BlockSpec((D, D), lambda i: (0, 0)),
            pl.BlockSpec((D, D), lambda i: (0, 0)),
            pl.BlockSpec((1, D), lambda i: (0, 0)),
        ],
        out_specs=pl.BlockSpec((_BR, D), lambda i: (i, 0)),
        out_shape=jax.ShapeDtypeStruct((NPAD, D), jnp.float32),
    )
    return fn(x, aggp, degp, w_self, w_neigh, b.reshape(1, D))


def kernel(in_feat, W_self1, W_neigh1, b1, W_self2, W_neigh2, b2, edge_index):
    src = edge_index[0].astype(jnp.int32)
    dst = edge_index[1].astype(jnp.int32)
    pad = EPAD - E
    src_r = jnp.concatenate([src, jnp.zeros((pad,), jnp.int32)]).reshape(NW, CH, C)
    dst_r = jnp.concatenate([dst, jnp.full((pad,), N, jnp.int32)]).reshape(NW, CH, C)
    x_pad = jnp.pad(in_feat, ((0, NPAD - N), (0, 0)))

    aggp1 = _sc_agg(x_pad, src_r, dst_r)
    degp1 = _sc_deg(dst_r)
    h1 = _tc_layer(x_pad, aggp1, degp1, W_self1, W_neigh1, b1, relu=True)
    aggp2 = _sc_agg(h1, src_r, dst_r)
    out = _tc_layer(h1, aggp2, degp1, W_self2, W_neigh2, b2, relu=False)
    return out[:N]
